# Initial kernel scaffold; baseline (speedup 1.0000x reference)
import jax
import jax.numpy as jnp
from jax.experimental import pallas as pl


def kernel(x, y, xstack0, xstack1, s0_embed_w, s0_embed_b, s0_ln1_g, s0_ln1_b, s0_ln2_g, s0_ln2_b, s0_mlp_w1, s0_mlp_b1, s0_mlp_w2, s0_mlp_b2, s1_embed_w, s1_embed_b, s1_ln1_g, s1_ln1_b, s1_ln2_g, s1_ln2_b, s1_mlp_w1, s1_mlp_b1, s1_mlp_w2, s1_mlp_b2, s2_embed_w, s2_embed_b, s2_ln1_g, s2_ln1_b, s2_ln2_g, s2_ln2_b, s2_mlp_w1, s2_mlp_b1, s2_mlp_w2, s2_mlp_b2, s3_embed_w, s3_embed_b, s3_ln1_g, s3_ln1_b, s3_ln2_g, s3_ln2_b, s3_mlp_w1, s3_mlp_b1, s3_mlp_w2, s3_mlp_b2, head_w, head_b):
    raise NotImplementedError("write your pallas kernel here")


# speedup vs baseline: 1.3517x; 1.3517x over previous
import functools

import numpy as np
import jax
import jax.numpy as jnp
from jax.experimental import pallas as pl
from jax.experimental.pallas import tpu as pltpu

B = 16
EMBED_DIMS = (32, 64, 128, 160)
OUT_HW = (256, 256)
_VMEM_LIMIT = 48 * 1024 * 1024


def _ln_f32(x, g, b, eps=1e-5):
    mu = jnp.mean(x, axis=-1, keepdims=True)
    xc = x - mu
    var = jnp.mean(xc * xc, axis=-1, keepdims=True)
    return xc * jax.lax.rsqrt(var + eps) * g + b


def _encoder_math(xs, ws, eb, g1, b1, g2, b2, w1, bb1, w2, bb2):
    tok = jnp.dot(xs[0], ws[0], preferred_element_type=jnp.float32)
    for xv, wv in zip(xs[1:], ws[1:]):
        tok = tok + jnp.dot(xv, wv, preferred_element_type=jnp.float32)
    tok = tok + eb
    tok = _ln_f32(tok, g1, b1)
    h = _ln_f32(tok, g2, b2)
    h = jnp.dot(h.astype(jnp.bfloat16), w1, preferred_element_type=jnp.float32) + bb1
    h = jax.nn.gelu(h, approximate=True)
    return (jnp.dot(h.astype(jnp.bfloat16), w2,
                    preferred_element_type=jnp.float32) + bb2 + tok)


def _regroup(out, hp, wp, c):
    o = out.reshape(hp // 2, 2, wp // 2, 2, c)
    parts = [o[:, dy, :, dx, :] for dy in (0, 1) for dx in (0, 1)]
    p = jnp.concatenate(parts, axis=-1)
    return p.reshape((hp * wp) // 4, 4 * c).astype(jnp.bfloat16)


def _stage0_body(pa_ref, pb_ref, pc_ref, wa_ref, wb_ref, wc_ref, eb_ref,
                 g1_ref, b1_ref, g2_ref, b2_ref, w1_ref, bb1_ref, w2_ref,
                 bb2_ref, o_ref, p_ref):
    out = _encoder_math(
        [pa_ref[...], pb_ref[...], pc_ref[...]],
        [wa_ref[...], wb_ref[...], wc_ref[...]],
        eb_ref[...], g1_ref[...], b1_ref[...], g2_ref[...], b2_ref[...],
        w1_ref[...], bb1_ref[...], w2_ref[...], bb2_ref[...])
    o_ref[...] = out.T.reshape(1, 32, 64, 64)
    p_ref[...] = _regroup(out, 64, 64, 32)


def _stage_body(p_in_ref, w_ref, eb_ref, g1_ref, b1_ref, g2_ref, b2_ref,
                w1_ref, bb1_ref, w2_ref, bb2_ref, o_ref, p_ref, *, hp, wp, c):
    out = _encoder_math(
        [p_in_ref[...]], [w_ref[...]],
        eb_ref[...], g1_ref[...], b1_ref[...], g2_ref[...], b2_ref[...],
        w1_ref[...], bb1_ref[...], w2_ref[...], bb2_ref[...])
    o_ref[...] = out.T.reshape(1, c, hp, wp)
    p_ref[...] = _regroup(out, hp, wp, c)


def _stage3_body(p_in_ref, w_ref, eb_ref, g1_ref, b1_ref, g2_ref, b2_ref,
                 w1_ref, bb1_ref, w2_ref, bb2_ref, o_ref, t_ref):
    out = _encoder_math(
        [p_in_ref[...]], [w_ref[...]],
        eb_ref[...], g1_ref[...], b1_ref[...], g2_ref[...], b2_ref[...],
        w1_ref[...], bb1_ref[...], w2_ref[...], bb2_ref[...])
    t = out.reshape(8, 64, 160)
    o_ref[...] = jnp.transpose(t, (0, 2, 1)).reshape(8, 160, 8, 8)
    t_ref[...] = out


def _full(shape):
    return pl.BlockSpec(shape, lambda i, _s=shape: tuple(0 for _ in _s))


def _stage_params(st, c):
    h = 4 * c
    return (st["embed_b"].reshape(1, c),
            st["ln1_g"].reshape(1, c), st["ln1_b"].reshape(1, c),
            st["ln2_g"].reshape(1, c), st["ln2_b"].reshape(1, c),
            st["mlp_w1"], st["mlp_b1"].reshape(1, h),
            st["mlp_w2"], st["mlp_b2"].reshape(1, c))


def _stage_param_specs(kin, c):
    h = 4 * c
    return [_full((1, c)), _full((1, c)), _full((1, c)), _full((1, c)),
            _full((1, c)), _full((c, h)), _full((1, h)), _full((h, c)),
            _full((1, c))]


def _compiler_params():
    return pltpu.CompilerParams(dimension_semantics=("parallel",),
                                vmem_limit_bytes=_VMEM_LIMIT)


def _bilinear_matrix_np(out_size, in_size):
    dst = np.arange(out_size, dtype=np.float32)
    scale = in_size / out_size
    src = np.clip((dst + 0.5) * scale - 0.5, 0.0, in_size - 1)
    i0 = np.floor(src).astype(np.int32)
    i1 = np.minimum(i0 + 1, in_size - 1)
    lam = (src - i0).astype(np.float32)
    a = np.zeros((out_size, in_size), np.float32)
    rows = np.arange(out_size)
    a[rows, i0] += 1.0 - lam
    a[rows, i1] += lam
    return a


@functools.lru_cache(maxsize=None)
def _bilinear_factors(h, w, oh, ow):
    a = jnp.asarray(_bilinear_matrix_np(oh, h))
    bt = jnp.asarray(_bilinear_matrix_np(ow, w).T)
    return a, bt


def _head_body(f_ref, hw_ref, hb_ref, a_ref, bt_ref, o_ref):
    feat = f_ref[0]
    hv = jnp.sum(feat * hw_ref[...], axis=-1, keepdims=True) + hb_ref[0, 0]
    img = hv.reshape(8, 8)
    tmp = jnp.dot(img, bt_ref[...], preferred_element_type=jnp.float32)
    o_ref[0] = jnp.dot(a_ref[...], tmp, preferred_element_type=jnp.float32)


def _s0_perm(g):
    idx = np.empty((48,), np.int32)
    k = 0
    for ch in range(3):
        for dy in range(4):
            for dx in range(4):
                idx[k] = (dy * 4 + dx) * 9 + 3 * g + ch
                k += 1
    return idx


def _patchify0(img):
    t = img.reshape(B, 3, 64, 4, 64, 4)
    t = jnp.transpose(t, (0, 2, 4, 1, 3, 5))
    return t.reshape(B * 4096, 48).astype(jnp.bfloat16)


def kernel(x, y, xstack0, xstack1,
           s0_embed_w, s0_embed_b, s0_ln1_g, s0_ln1_b, s0_ln2_g, s0_ln2_b,
           s0_mlp_w1, s0_mlp_b1, s0_mlp_w2, s0_mlp_b2,
           s1_embed_w, s1_embed_b, s1_ln1_g, s1_ln1_b, s1_ln2_g, s1_ln2_b,
           s1_mlp_w1, s1_mlp_b1, s1_mlp_w2, s1_mlp_b2,
           s2_embed_w, s2_embed_b, s2_ln1_g, s2_ln1_b, s2_ln2_g, s2_ln2_b,
           s2_mlp_w1, s2_mlp_b1, s2_mlp_w2, s2_mlp_b2,
           s3_embed_w, s3_embed_b, s3_ln1_g, s3_ln1_b, s3_ln2_g, s3_ln2_b,
           s3_mlp_w1, s3_mlp_b1, s3_mlp_w2, s3_mlp_b2,
           head_w, head_b):
    stages = [
        dict(embed_w=s0_embed_w, embed_b=s0_embed_b, ln1_g=s0_ln1_g,
             ln1_b=s0_ln1_b, ln2_g=s0_ln2_g, ln2_b=s0_ln2_b,
             mlp_w1=s0_mlp_w1, mlp_b1=s0_mlp_b1, mlp_w2=s0_mlp_w2,
             mlp_b2=s0_mlp_b2),
        dict(embed_w=s1_embed_w, embed_b=s1_embed_b, ln1_g=s1_ln1_g,
             ln1_b=s1_ln1_b, ln2_g=s1_ln2_g, ln2_b=s1_ln2_b,
             mlp_w1=s1_mlp_w1, mlp_b1=s1_mlp_b1, mlp_w2=s1_mlp_w2,
             mlp_b2=s1_mlp_b2),
        dict(embed_w=s2_embed_w, embed_b=s2_embed_b, ln1_g=s2_ln1_g,
             ln1_b=s2_ln1_b, ln2_g=s2_ln2_g, ln2_b=s2_ln2_b,
             mlp_w1=s2_mlp_w1, mlp_b1=s2_mlp_b1, mlp_w2=s2_mlp_w2,
             mlp_b2=s2_mlp_b2),
        dict(embed_w=s3_embed_w, embed_b=s3_embed_b, ln1_g=s3_ln1_g,
             ln1_b=s3_ln1_b, ln2_g=s3_ln2_g, ln2_b=s3_ln2_b,
             mlp_w1=s3_mlp_w1, mlp_b1=s3_mlp_b1, mlp_w2=s3_mlp_w2,
             mlp_b2=s3_mlp_b2),
    ]

    pa = _patchify0(xstack0)
    pb = _patchify0(xstack1)
    pc = _patchify0(y)
    wa = s0_embed_w[jnp.asarray(_s0_perm(0))]
    wb = s0_embed_w[jnp.asarray(_s0_perm(1))]
    wc = s0_embed_w[jnp.asarray(_s0_perm(2))]

    st = stages[0]
    out1, p1 = pl.pallas_call(
        _stage0_body,
        out_shape=(jax.ShapeDtypeStruct((B, 32, 64, 64), jnp.float32),
                   jax.ShapeDtypeStruct((B * 1024, 128), jnp.bfloat16)),
        grid=(B,),
        in_specs=[
            pl.BlockSpec((4096, 48), lambda b: (b, 0)),
            pl.BlockSpec((4096, 48), lambda b: (b, 0)),
            pl.BlockSpec((4096, 48), lambda b: (b, 0)),
            _full((48, 32)), _full((48, 32)), _full((48, 32)),
        ] + _stage_param_specs(144, 32),
        out_specs=(pl.BlockSpec((1, 32, 64, 64), lambda b: (b, 0, 0, 0)),
                   pl.BlockSpec((1024, 128), lambda b: (b, 0))),
        compiler_params=_compiler_params(),
    )(pa, pb, pc, wa, wb, wc, *_stage_params(st, 32))

    st = stages[1]
    out2, p2 = pl.pallas_call(
        functools.partial(_stage_body, hp=32, wp=32, c=64),
        out_shape=(jax.ShapeDtypeStruct((B, 64, 32, 32), jnp.float32),
                   jax.ShapeDtypeStruct((B * 256, 256), jnp.bfloat16)),
        grid=(B,),
        in_specs=[pl.BlockSpec((1024, 128), lambda b: (b, 0)),
                  _full((128, 64))] + _stage_param_specs(128, 64),
        out_specs=(pl.BlockSpec((1, 64, 32, 32), lambda b: (b, 0, 0, 0)),
                   pl.BlockSpec((256, 256), lambda b: (b, 0))),
        compiler_params=_compiler_params(),
    )(p1, st["embed_w"], *_stage_params(st, 64))

    st = stages[2]
    out3, p3 = pl.pallas_call(
        functools.partial(_stage_body, hp=16, wp=16, c=128),
        out_shape=(jax.ShapeDtypeStruct((B, 128, 16, 16), jnp.float32),
                   jax.ShapeDtypeStruct((B * 64, 512), jnp.bfloat16)),
        grid=(B,),
        in_specs=[pl.BlockSpec((256, 256), lambda b: (b, 0)),
                  _full((256, 128))] + _stage_param_specs(256, 128),
        out_specs=(pl.BlockSpec((1, 128, 16, 16), lambda b: (b, 0, 0, 0)),
                   pl.BlockSpec((64, 512), lambda b: (b, 0))),
        compiler_params=_compiler_params(),
    )(p2, st["embed_w"], *_stage_params(st, 128))

    st = stages[3]
    out4, tok4 = pl.pallas_call(
        _stage3_body,
        out_shape=(jax.ShapeDtypeStruct((B, 160, 8, 8), jnp.float32),
                   jax.ShapeDtypeStruct((B * 64, 160), jnp.float32)),
        grid=(2,),
        in_specs=[pl.BlockSpec((512, 512), lambda i: (i, 0)),
                  _full((512, 160))] + _stage_param_specs(512, 160),
        out_specs=(pl.BlockSpec((8, 160, 8, 8), lambda i: (i, 0, 0, 0)),
                   pl.BlockSpec((512, 160), lambda i: (i, 0))),
        compiler_params=_compiler_params(),
    )(p3, st["embed_w"], *_stage_params(st, 160))

    oh, ow = OUT_HW
    a_mat, bt_mat = _bilinear_factors(8, 8, oh, ow)
    feat = tok4.reshape(B, 64, 160)
    rgb = pl.pallas_call(
        _head_body,
        out_shape=jax.ShapeDtypeStruct((B, oh, ow), jnp.float32),
        grid=(B,),
        in_specs=[
            pl.BlockSpec((1, 64, 160), lambda b: (b, 0, 0)),
            _full((1, 160)), _full((1, 1)),
            _full((oh, 8)), _full((8, ow)),
        ],
        out_specs=pl.BlockSpec((1, oh, ow), lambda b: (b, 0, 0)),
        compiler_params=_compiler_params(),
    )(feat, head_w.reshape(1, 160), head_b.reshape(1, 1), a_mat, bt_mat)

    return rgb.reshape(B, 1, oh, ow), out1, out2, out3, out4
